# baseline (speedup 1.0000x reference)
import numpy as np
import jax
import jax.numpy as jnp
from jax.experimental import pallas as pl
from jax.experimental.pallas import tpu as pltpu

_B, _S, _D = 1, 2048, 1024
_H, _DH = 16, 64
_BS, _R = 64, 3
_NB = _S // _BS
_INNER = 1024
_KMAX = 8
_NEG = -1e9


def _plan():
    rng = np.random.default_rng(0)
    m = np.zeros((_NB, _NB), dtype=bool)
    m[0, :] = True
    m[-1, :] = True
    m[:, 0] = True
    m[:, -1] = True
    for i in range(_NB):
        for j in (i - 1, i, i + 1):
            if 0 <= j < _NB:
                m[i, j] = True
    for i in range(1, _NB - 1):
        m[i, rng.choice(np.arange(1, _NB - 1), size=_R, replace=False)] = True
    tbl = np.zeros((_NB, _KMAX), np.int32)
    cnt = np.zeros((_NB,), np.int32)
    for i in range(1, _NB - 1):
        js = np.nonzero(m[i])[0]
        assert len(js) <= _KMAX
        cnt[i] = len(js)
        tbl[i, : len(js)] = js
    return tbl, cnt


_TBL_NP, _CNT_NP = _plan()


def _qkv_body(x_ref, w_ref, b_ref, q_ref, k_ref, v_ref):
    y = jnp.dot(x_ref[...], w_ref[...], preferred_element_type=jnp.float32)
    y = y + b_ref[...]
    q_ref[...] = y[:, :_D]
    k_ref[...] = y[:, _D:2 * _D]
    v_ref[...] = y[:, 2 * _D:]


def _attn_body(tbl_ref, cnt_ref, q_ref, k_ref, v_ref, kbf_ref, kbb_ref,
               o_ref, s_scr):
    i = pl.program_id(1)
    q = q_ref[0]
    scale = 0.125
    edge = jnp.logical_or(i == 0, i == _NB - 1)

    @pl.when(edge)
    def _():
        s = jax.lax.dot_general(q, k_ref[0], (((1,), (1,)), ((), ())),
                                preferred_element_type=jnp.float32)
        s = s * scale + kbf_ref[...]
        m = jnp.max(s, axis=1, keepdims=True)
        p = jnp.exp(s - m)
        denom = jnp.sum(p, axis=1, keepdims=True)
        ctx = jax.lax.dot_general(p, v_ref[0], (((1,), (0,)), ((), ())),
                                  preferred_element_type=jnp.float32)
        o_ref[0] = ctx / denom

    @pl.when(jnp.logical_not(edge))
    def _():
        cnt = cnt_ref[i]
        m = jnp.full((_BS, 1), _NEG, jnp.float32)
        for j in range(_KMAX):
            kb = tbl_ref[i, j]
            kblk = k_ref[0, pl.ds(kb * _BS, _BS), :]
            s = jax.lax.dot_general(q, kblk, (((1,), (1,)), ((), ())),
                                    preferred_element_type=jnp.float32)
            s = s * scale + kbb_ref[pl.ds(kb, 1), :]
            pad = jnp.where(j < cnt, 0.0, _NEG).astype(jnp.float32)
            s = s + pad
            s_scr[j] = s
            m = jnp.maximum(m, jnp.max(s, axis=1, keepdims=True))
        denom = jnp.zeros((_BS, 1), jnp.float32)
        acc = jnp.zeros((_BS, _DH), jnp.float32)
        for j in range(_KMAX):
            kb = tbl_ref[i, j]
            p = jnp.exp(s_scr[j] - m)
            denom = denom + jnp.sum(p, axis=1, keepdims=True)
            vblk = v_ref[0, pl.ds(kb * _BS, _BS), :]
            acc = acc + jax.lax.dot_general(p, vblk, (((1,), (0,)), ((), ())),
                                            preferred_element_type=jnp.float32)
        o_ref[0] = acc / denom


def _ln_tile(x, g, b, eps=1e-12):
    mu = jnp.mean(x, axis=1, keepdims=True)
    xc = x - mu
    var = jnp.mean(xc * xc, axis=1, keepdims=True)
    return xc * jax.lax.rsqrt(var + eps) * g + b


def _post_body(ctx_ref, x_ref, wo_ref, bo_ref, g1_ref, b1_ref,
               wi_ref, bi_ref, wo2_ref, bo2_ref, g2_ref, b2_ref, o_ref):
    attn = jnp.dot(ctx_ref[...], wo_ref[...],
                   preferred_element_type=jnp.float32) + bo_ref[...]
    s = _ln_tile(attn + x_ref[...], g1_ref[...], b1_ref[...])
    inner = jnp.dot(s, wi_ref[...], preferred_element_type=jnp.float32)
    inner = jnp.maximum(inner + bi_ref[...], 0.0)
    o = jnp.dot(inner, wo2_ref[...],
                preferred_element_type=jnp.float32) + bo2_ref[...]
    o_ref[...] = _ln_tile(o + s, g2_ref[...], b2_ref[...])


_ROWB = 256
_NROW = _S // _ROWB


def kernel(content_stream, mask, Wq, bq, Wk, bk, Wv, bv, Wo, bo,
           g1, b1, Wi, bi, Wo2, bo2, g2, b2):
    x = content_stream.reshape(_S, _D)
    wqkv = jnp.concatenate(
        [Wq.reshape(_D, _H * _DH), Wk.reshape(_D, _H * _DH),
         Wv.reshape(_D, _H * _DH)], axis=1)
    bqkv = jnp.concatenate(
        [bq.reshape(-1), bk.reshape(-1), bv.reshape(-1)])[None, :]

    q2, k2, v2 = pl.pallas_call(
        _qkv_body,
        grid=(_NROW,),
        in_specs=[
            pl.BlockSpec((_ROWB, _D), lambda r: (r, 0)),
            pl.BlockSpec((_D, 3 * _D), lambda r: (0, 0)),
            pl.BlockSpec((1, 3 * _D), lambda r: (0, 0)),
        ],
        out_specs=[
            pl.BlockSpec((_ROWB, _D), lambda r: (r, 0)),
            pl.BlockSpec((_ROWB, _D), lambda r: (r, 0)),
            pl.BlockSpec((_ROWB, _D), lambda r: (r, 0)),
        ],
        out_shape=[jax.ShapeDtypeStruct((_S, _D), jnp.float32)] * 3,
    )(x, wqkv, bqkv)

    q3 = q2.reshape(_S, _H, _DH).transpose(1, 0, 2)
    k3 = k2.reshape(_S, _H, _DH).transpose(1, 0, 2)
    v3 = v2.reshape(_S, _H, _DH).transpose(1, 0, 2)

    kbias = jnp.where(mask.reshape(_S) > 0, 0.0, _NEG).astype(jnp.float32)
    kbias_flat = kbias[None, :]
    kbias_blk = kbias.reshape(_NB, _BS)
    tbl = jnp.asarray(_TBL_NP)
    cnt = jnp.asarray(_CNT_NP)

    ctx3 = pl.pallas_call(
        _attn_body,
        grid=(_H, _NB),
        in_specs=[
            pl.BlockSpec(memory_space=pltpu.SMEM),
            pl.BlockSpec(memory_space=pltpu.SMEM),
            pl.BlockSpec((1, _BS, _DH), lambda h, i: (h, i, 0)),
            pl.BlockSpec((1, _S, _DH), lambda h, i: (h, 0, 0)),
            pl.BlockSpec((1, _S, _DH), lambda h, i: (h, 0, 0)),
            pl.BlockSpec((1, _S), lambda h, i: (0, 0)),
            pl.BlockSpec((_NB, _BS), lambda h, i: (0, 0)),
        ],
        out_specs=pl.BlockSpec((1, _BS, _DH), lambda h, i: (h, i, 0)),
        out_shape=jax.ShapeDtypeStruct((_H, _S, _DH), jnp.float32),
        scratch_shapes=[pltpu.VMEM((_KMAX, _BS, _BS), jnp.float32)],
    )(tbl, cnt, q3, k3, v3, kbias_flat, kbias_blk)

    ctx = ctx3.transpose(1, 0, 2).reshape(_S, _H * _DH)

    out = pl.pallas_call(
        _post_body,
        grid=(_NROW,),
        in_specs=[
            pl.BlockSpec((_ROWB, _D), lambda r: (r, 0)),
            pl.BlockSpec((_ROWB, _D), lambda r: (r, 0)),
            pl.BlockSpec((_H * _DH, _D), lambda r: (0, 0)),
            pl.BlockSpec((1, _D), lambda r: (0, 0)),
            pl.BlockSpec((1, _D), lambda r: (0, 0)),
            pl.BlockSpec((1, _D), lambda r: (0, 0)),
            pl.BlockSpec((_D, _INNER), lambda r: (0, 0)),
            pl.BlockSpec((1, _INNER), lambda r: (0, 0)),
            pl.BlockSpec((_INNER, _INNER), lambda r: (0, 0)),
            pl.BlockSpec((1, _INNER), lambda r: (0, 0)),
            pl.BlockSpec((1, _INNER), lambda r: (0, 0)),
            pl.BlockSpec((1, _INNER), lambda r: (0, 0)),
        ],
        out_specs=pl.BlockSpec((_ROWB, _INNER), lambda r: (r, 0)),
        out_shape=jax.ShapeDtypeStruct((_S, _INNER), jnp.float32),
    )(ctx, x, Wo.reshape(_H * _DH, _D), bo[None, :], g1[None, :], b1[None, :],
      Wi, bi[None, :], Wo2, bo2[None, :], g2[None, :], b2[None, :])

    return out.reshape(_B, _S, _INNER)


# speedup vs baseline: 2.2314x; 2.2314x over previous
import numpy as np
import jax
import jax.numpy as jnp
from jax.experimental import pallas as pl
from jax.experimental.pallas import tpu as pltpu

_B, _S, _D = 1, 2048, 1024
_H, _DH = 16, 64
_BS, _R = 64, 3
_NB = _S // _BS
_INNER = 1024
_KMAX = 8
_NEG = -1e9


def _plan():
    rng = np.random.default_rng(0)
    m = np.zeros((_NB, _NB), dtype=bool)
    m[0, :] = True
    m[-1, :] = True
    m[:, 0] = True
    m[:, -1] = True
    for i in range(_NB):
        for j in (i - 1, i, i + 1):
            if 0 <= j < _NB:
                m[i, j] = True
    for i in range(1, _NB - 1):
        m[i, rng.choice(np.arange(1, _NB - 1), size=_R, replace=False)] = True
    tbl = np.zeros((_NB, _KMAX), np.int32)
    valid = np.zeros((_NB, _KMAX), np.bool_)
    for i in range(1, _NB - 1):
        js = np.nonzero(m[i])[0]
        assert len(js) <= _KMAX
        tbl[i, : len(js)] = js
        valid[i, : len(js)] = True
    return tbl, valid


_TBL_NP, _VALID_NP = _plan()


def _qkv_body(x_ref, w_ref, b_ref, q_ref, k_ref, v_ref):
    y = jnp.dot(x_ref[...], w_ref[...], preferred_element_type=jnp.float32)
    y = y + b_ref[...]
    q_ref[...] = y[:, :_D]
    k_ref[...] = y[:, _D:2 * _D]
    v_ref[...] = y[:, 2 * _D:]


def _attn_body(tbl_ref, q_ref, k_ref, v_ref, kbf_ref, sb_ref, o_ref):
    i = pl.program_id(0)
    scale = 0.125
    edge = jnp.logical_or(i == 0, i == _NB - 1)

    @pl.when(edge)
    def _():
        for h in range(_H):
            lo, hi = h * _DH, (h + 1) * _DH
            qh = q_ref[:, lo:hi]
            s = jax.lax.dot_general(qh, k_ref[:, lo:hi],
                                    (((1,), (1,)), ((), ())),
                                    preferred_element_type=jnp.float32)
            s = s * scale + kbf_ref[...]
            m = jnp.max(s, axis=1, keepdims=True)
            p = jnp.exp(s - m)
            denom = jnp.sum(p, axis=1, keepdims=True)
            ctx = jax.lax.dot_general(p, v_ref[:, lo:hi],
                                      (((1,), (0,)), ((), ())),
                                      preferred_element_type=jnp.float32)
            o_ref[:, lo:hi] = ctx / denom

    @pl.when(jnp.logical_not(edge))
    def _():
        sbias = sb_ref[0]
        for h in range(_H):
            lo, hi = h * _DH, (h + 1) * _DH
            qh = q_ref[:, lo:hi]
            kcat = jnp.concatenate(
                [k_ref[pl.ds(tbl_ref[i, j] * _BS, _BS), lo:hi]
                 for j in range(_KMAX)], axis=0)
            s = jax.lax.dot_general(qh, kcat, (((1,), (1,)), ((), ())),
                                    preferred_element_type=jnp.float32)
            s = s * scale + sbias
            m = jnp.max(s, axis=1, keepdims=True)
            p = jnp.exp(s - m)
            denom = jnp.sum(p, axis=1, keepdims=True)
            vcat = jnp.concatenate(
                [v_ref[pl.ds(tbl_ref[i, j] * _BS, _BS), lo:hi]
                 for j in range(_KMAX)], axis=0)
            ctx = jax.lax.dot_general(p, vcat, (((1,), (0,)), ((), ())),
                                      preferred_element_type=jnp.float32)
            o_ref[:, lo:hi] = ctx / denom


def _ln_tile(x, g, b, eps=1e-12):
    mu = jnp.mean(x, axis=1, keepdims=True)
    xc = x - mu
    var = jnp.mean(xc * xc, axis=1, keepdims=True)
    return xc * jax.lax.rsqrt(var + eps) * g + b


def _post_body(ctx_ref, x_ref, wo_ref, bo_ref, g1_ref, b1_ref,
               wi_ref, bi_ref, wo2_ref, bo2_ref, g2_ref, b2_ref, o_ref):
    attn = jnp.dot(ctx_ref[...], wo_ref[...],
                   preferred_element_type=jnp.float32) + bo_ref[...]
    s = _ln_tile(attn + x_ref[...], g1_ref[...], b1_ref[...])
    inner = jnp.dot(s, wi_ref[...], preferred_element_type=jnp.float32)
    inner = jnp.maximum(inner + bi_ref[...], 0.0)
    o = jnp.dot(inner, wo2_ref[...],
                preferred_element_type=jnp.float32) + bo2_ref[...]
    o_ref[...] = _ln_tile(o + s, g2_ref[...], b2_ref[...])


_ROWB = 256
_NROW = _S // _ROWB


def kernel(content_stream, mask, Wq, bq, Wk, bk, Wv, bv, Wo, bo,
           g1, b1, Wi, bi, Wo2, bo2, g2, b2):
    x = content_stream.reshape(_S, _D)
    wqkv = jnp.concatenate(
        [Wq.reshape(_D, _H * _DH), Wk.reshape(_D, _H * _DH),
         Wv.reshape(_D, _H * _DH)], axis=1)
    bqkv = jnp.concatenate(
        [bq.reshape(-1), bk.reshape(-1), bv.reshape(-1)])[None, :]

    q2, k2, v2 = pl.pallas_call(
        _qkv_body,
        grid=(_NROW,),
        in_specs=[
            pl.BlockSpec((_ROWB, _D), lambda r: (r, 0)),
            pl.BlockSpec((_D, 3 * _D), lambda r: (0, 0)),
            pl.BlockSpec((1, 3 * _D), lambda r: (0, 0)),
        ],
        out_specs=[
            pl.BlockSpec((_ROWB, _D), lambda r: (r, 0)),
            pl.BlockSpec((_ROWB, _D), lambda r: (r, 0)),
            pl.BlockSpec((_ROWB, _D), lambda r: (r, 0)),
        ],
        out_shape=[jax.ShapeDtypeStruct((_S, _D), jnp.float32)] * 3,
    )(x, wqkv, bqkv)

    kbias = jnp.where(mask.reshape(_S) > 0, 0.0, _NEG).astype(jnp.float32)
    kbias_flat = kbias[None, :]
    tbl = jnp.asarray(_TBL_NP)
    sbias = jnp.where(jnp.asarray(_VALID_NP)[:, :, None], 0.0, _NEG)
    sbias = sbias + kbias.reshape(_NB, _BS)[tbl]
    sbias = sbias.reshape(_NB, 1, _KMAX * _BS).astype(jnp.float32)

    ctx = pl.pallas_call(
        _attn_body,
        grid=(_NB,),
        in_specs=[
            pl.BlockSpec(memory_space=pltpu.SMEM),
            pl.BlockSpec((_BS, _D), lambda i: (i, 0)),
            pl.BlockSpec((_S, _D), lambda i: (0, 0)),
            pl.BlockSpec((_S, _D), lambda i: (0, 0)),
            pl.BlockSpec((1, _S), lambda i: (0, 0)),
            pl.BlockSpec((1, 1, _KMAX * _BS), lambda i: (i, 0, 0)),
        ],
        out_specs=pl.BlockSpec((_BS, _D), lambda i: (i, 0)),
        out_shape=jax.ShapeDtypeStruct((_S, _D), jnp.float32),
    )(tbl, q2, k2, v2, kbias_flat, sbias)

    out = pl.pallas_call(
        _post_body,
        grid=(_NROW,),
        in_specs=[
            pl.BlockSpec((_ROWB, _D), lambda r: (r, 0)),
            pl.BlockSpec((_ROWB, _D), lambda r: (r, 0)),
            pl.BlockSpec((_H * _DH, _D), lambda r: (0, 0)),
            pl.BlockSpec((1, _D), lambda r: (0, 0)),
            pl.BlockSpec((1, _D), lambda r: (0, 0)),
            pl.BlockSpec((1, _D), lambda r: (0, 0)),
            pl.BlockSpec((_D, _INNER), lambda r: (0, 0)),
            pl.BlockSpec((1, _INNER), lambda r: (0, 0)),
            pl.BlockSpec((_INNER, _INNER), lambda r: (0, 0)),
            pl.BlockSpec((1, _INNER), lambda r: (0, 0)),
            pl.BlockSpec((1, _INNER), lambda r: (0, 0)),
            pl.BlockSpec((1, _INNER), lambda r: (0, 0)),
        ],
        out_specs=pl.BlockSpec((_ROWB, _INNER), lambda r: (r, 0)),
        out_shape=jax.ShapeDtypeStruct((_S, _INNER), jnp.float32),
    )(ctx, x, Wo.reshape(_H * _DH, _D), bo[None, :], g1[None, :], b1[None, :],
      Wi, bi[None, :], Wo2, bo2[None, :], g2[None, :], b2[None, :])

    return out.reshape(_B, _S, _INNER)


# speedup vs baseline: 2.2729x; 1.0186x over previous
import numpy as np
import jax
import jax.numpy as jnp
from jax.experimental import pallas as pl
from jax.experimental.pallas import tpu as pltpu

_B, _S, _D = 1, 2048, 1024
_H, _DH = 16, 64
_BS, _R = 64, 3
_NB = _S // _BS
_INNER = 1024
_KMAX = 8
_NEG = -1e9


def _plan():
    rng = np.random.default_rng(0)
    m = np.zeros((_NB, _NB), dtype=bool)
    m[0, :] = True
    m[-1, :] = True
    m[:, 0] = True
    m[:, -1] = True
    for i in range(_NB):
        for j in (i - 1, i, i + 1):
            if 0 <= j < _NB:
                m[i, j] = True
    for i in range(1, _NB - 1):
        m[i, rng.choice(np.arange(1, _NB - 1), size=_R, replace=False)] = True
    tbl = np.zeros((_NB, _KMAX), np.int32)
    valid = np.zeros((_NB, _KMAX), np.bool_)
    for i in range(1, _NB - 1):
        js = np.nonzero(m[i])[0]
        assert len(js) <= _KMAX
        tbl[i, : len(js)] = js
        valid[i, : len(js)] = True
    return tbl, valid


_TBL_NP, _VALID_NP = _plan()


def _qkv_body(x_ref, w_ref, b_ref, q_ref, k_ref, v_ref):
    y = jnp.dot(x_ref[...], w_ref[...], preferred_element_type=jnp.float32)
    y = y + b_ref[...]
    q_ref[...] = y[:, :_D]
    k_ref[...] = y[:, _D:2 * _D]
    v_ref[...] = y[:, 2 * _D:]


def _attn_body(tbl_ref, q_ref, k_ref, v_ref, kbf_ref, sb_ref, o_ref,
               kbuf, vbuf):
    i = pl.program_id(0)
    scale = 0.125
    edge = jnp.logical_or(i == 0, i == _NB - 1)

    @pl.when(edge)
    def _():
        for h in range(_H):
            lo, hi = h * _DH, (h + 1) * _DH
            qh = q_ref[:, lo:hi]
            s = jax.lax.dot_general(qh, k_ref[:, lo:hi],
                                    (((1,), (1,)), ((), ())),
                                    preferred_element_type=jnp.float32)
            s = s * scale + kbf_ref[...]
            m = jnp.max(s, axis=1, keepdims=True)
            p = jnp.exp(s - m)
            denom = jnp.sum(p, axis=1, keepdims=True)
            ctx = jax.lax.dot_general(p, v_ref[:, lo:hi],
                                      (((1,), (0,)), ((), ())),
                                      preferred_element_type=jnp.float32)
            o_ref[:, lo:hi] = ctx / denom

    @pl.when(jnp.logical_not(edge))
    def _():
        sbias = sb_ref[0]
        for j in range(_KMAX):
            kb = tbl_ref[i, j]
            kbuf[j * _BS:(j + 1) * _BS, :] = k_ref[pl.ds(kb * _BS, _BS), :]
            vbuf[j * _BS:(j + 1) * _BS, :] = v_ref[pl.ds(kb * _BS, _BS), :]
        for h in range(_H):
            lo, hi = h * _DH, (h + 1) * _DH
            qh = q_ref[:, lo:hi]
            s = jax.lax.dot_general(qh, kbuf[:, lo:hi],
                                    (((1,), (1,)), ((), ())),
                                    preferred_element_type=jnp.float32)
            s = s * scale + sbias
            m = jnp.max(s, axis=1, keepdims=True)
            p = jnp.exp(s - m)
            denom = jnp.sum(p, axis=1, keepdims=True)
            ctx = jax.lax.dot_general(p, vbuf[:, lo:hi],
                                      (((1,), (0,)), ((), ())),
                                      preferred_element_type=jnp.float32)
            o_ref[:, lo:hi] = ctx / denom


def _ln_tile(x, g, b, eps=1e-12):
    mu = jnp.mean(x, axis=1, keepdims=True)
    xc = x - mu
    var = jnp.mean(xc * xc, axis=1, keepdims=True)
    return xc * jax.lax.rsqrt(var + eps) * g + b


def _post_body(ctx_ref, x_ref, wo_ref, bo_ref, g1_ref, b1_ref,
               wi_ref, bi_ref, wo2_ref, bo2_ref, g2_ref, b2_ref, o_ref):
    attn = jnp.dot(ctx_ref[...], wo_ref[...],
                   preferred_element_type=jnp.float32) + bo_ref[...]
    s = _ln_tile(attn + x_ref[...], g1_ref[...], b1_ref[...])
    inner = jnp.dot(s, wi_ref[...], preferred_element_type=jnp.float32)
    inner = jnp.maximum(inner + bi_ref[...], 0.0)
    o = jnp.dot(inner, wo2_ref[...],
                preferred_element_type=jnp.float32) + bo2_ref[...]
    o_ref[...] = _ln_tile(o + s, g2_ref[...], b2_ref[...])


_ROWB = 256
_NROW = _S // _ROWB


def kernel(content_stream, mask, Wq, bq, Wk, bk, Wv, bv, Wo, bo,
           g1, b1, Wi, bi, Wo2, bo2, g2, b2):
    x = content_stream.reshape(_S, _D)
    wqkv = jnp.concatenate(
        [Wq.reshape(_D, _H * _DH), Wk.reshape(_D, _H * _DH),
         Wv.reshape(_D, _H * _DH)], axis=1)
    bqkv = jnp.concatenate(
        [bq.reshape(-1), bk.reshape(-1), bv.reshape(-1)])[None, :]

    q2, k2, v2 = pl.pallas_call(
        _qkv_body,
        grid=(_NROW,),
        in_specs=[
            pl.BlockSpec((_ROWB, _D), lambda r: (r, 0)),
            pl.BlockSpec((_D, 3 * _D), lambda r: (0, 0)),
            pl.BlockSpec((1, 3 * _D), lambda r: (0, 0)),
        ],
        out_specs=[
            pl.BlockSpec((_ROWB, _D), lambda r: (r, 0)),
            pl.BlockSpec((_ROWB, _D), lambda r: (r, 0)),
            pl.BlockSpec((_ROWB, _D), lambda r: (r, 0)),
        ],
        out_shape=[jax.ShapeDtypeStruct((_S, _D), jnp.float32)] * 3,
    )(x, wqkv, bqkv)

    kbias = jnp.where(mask.reshape(_S) > 0, 0.0, _NEG).astype(jnp.float32)
    kbias_flat = kbias[None, :]
    tbl = jnp.asarray(_TBL_NP)
    sbias = jnp.where(jnp.asarray(_VALID_NP)[:, :, None], 0.0, _NEG)
    sbias = sbias + kbias.reshape(_NB, _BS)[tbl]
    sbias = sbias.reshape(_NB, 1, _KMAX * _BS).astype(jnp.float32)

    ctx = pl.pallas_call(
        _attn_body,
        grid=(_NB,),
        in_specs=[
            pl.BlockSpec(memory_space=pltpu.SMEM),
            pl.BlockSpec((_BS, _D), lambda i: (i, 0)),
            pl.BlockSpec((_S, _D), lambda i: (0, 0)),
            pl.BlockSpec((_S, _D), lambda i: (0, 0)),
            pl.BlockSpec((1, _S), lambda i: (0, 0)),
            pl.BlockSpec((1, 1, _KMAX * _BS), lambda i: (i, 0, 0)),
        ],
        out_specs=pl.BlockSpec((_BS, _D), lambda i: (i, 0)),
        out_shape=jax.ShapeDtypeStruct((_S, _D), jnp.float32),
        scratch_shapes=[pltpu.VMEM((_KMAX * _BS, _D), jnp.float32),
                        pltpu.VMEM((_KMAX * _BS, _D), jnp.float32)],
    )(tbl, q2, k2, v2, kbias_flat, sbias)

    out = pl.pallas_call(
        _post_body,
        grid=(_NROW,),
        in_specs=[
            pl.BlockSpec((_ROWB, _D), lambda r: (r, 0)),
            pl.BlockSpec((_ROWB, _D), lambda r: (r, 0)),
            pl.BlockSpec((_H * _DH, _D), lambda r: (0, 0)),
            pl.BlockSpec((1, _D), lambda r: (0, 0)),
            pl.BlockSpec((1, _D), lambda r: (0, 0)),
            pl.BlockSpec((1, _D), lambda r: (0, 0)),
            pl.BlockSpec((_D, _INNER), lambda r: (0, 0)),
            pl.BlockSpec((1, _INNER), lambda r: (0, 0)),
            pl.BlockSpec((_INNER, _INNER), lambda r: (0, 0)),
            pl.BlockSpec((1, _INNER), lambda r: (0, 0)),
            pl.BlockSpec((1, _INNER), lambda r: (0, 0)),
            pl.BlockSpec((1, _INNER), lambda r: (0, 0)),
        ],
        out_specs=pl.BlockSpec((_ROWB, _INNER), lambda r: (r, 0)),
        out_shape=jax.ShapeDtypeStruct((_S, _INNER), jnp.float32),
    )(ctx, x, Wo.reshape(_H * _DH, _D), bo[None, :], g1[None, :], b1[None, :],
      Wi, bi[None, :], Wo2, bo2[None, :], g2[None, :], b2[None, :])

    return out.reshape(_B, _S, _INNER)
